# Initial kernel scaffold; baseline (speedup 1.0000x reference)
import jax
import jax.numpy as jnp
from jax.experimental import pallas as pl


def kernel(x, params, edge_index):
    raise NotImplementedError("write your pallas kernel here")


# speedup vs baseline: 80.9545x; 80.9545x over previous
import functools
import math

import jax
import jax.numpy as jnp
from jax import lax
from jax.experimental import pallas as pl
from jax.experimental.pallas import tpu as pltpu

B, N, E, H, NL = 64, 600, 9600, 128, 4
NP = 640
EPS = 1e-5
_INV = 1.0 / math.sqrt(1.0 + EPS)
_CHUNKS = [(c * 512, 512) for c in range(18)] + [(18 * 512, 384)]


def _f32(x):
    return x.astype(jnp.float32)


def _gnn_kernel(xT, edges, W1, b1, W2, b2, W3, b3,
                Wsn0, bsn0, bn0, g0, bg0,
                Wsn1, bsn1, bn1, g1, bg1,
                Wsn2, bsn2, bn2, g2, bg2,
                Wsn3, bsn3, bn3, g3, bg3,
                Wd1, bd1, Wd2, bd2, w3r, bd3,
                out, A_scr):
    pid = pl.program_id(0)

    @pl.when(pid == 0)
    def _build():
        A_scr[...] = jnp.zeros((NP, NP), jnp.float32)
        for base, sz in _CHUNKS:
            dst = edges[1:2, pl.ds(base, sz)]
            src = edges[0:1, pl.ds(base, sz)]
            rows = lax.broadcasted_iota(jnp.int32, (NP, sz), 0)
            dst_oh = _f32(rows == dst)
            src_oh = _f32(rows == src)
            A_scr[...] += lax.dot_general(
                dst_oh, src_oh, (((1,), (1,)), ((), ())),
                preferred_element_type=jnp.float32)

    A = A_scr[...]
    deg = jnp.sum(A, axis=1, keepdims=True)
    invd = 1.0 / jnp.maximum(deg, 1.0)
    dscale = deg * invd

    sel = _f32(lax.broadcasted_iota(jnp.int32, (1, B), 1) == pid)
    xc = jnp.sum(xT[...] * sel, axis=1, keepdims=True)

    h = jax.nn.relu(xc * W1[...] + b1[...])
    h = jax.nn.relu(jnp.dot(h, W2[...], preferred_element_type=jnp.float32)
                    + b2[...])
    h = jnp.dot(h, W3[...], preferred_element_type=jnp.float32) + b3[...]

    layer_refs = ((Wsn0, bsn0, bn0, g0, bg0), (Wsn1, bsn1, bn1, g1, bg1),
                  (Wsn2, bsn2, bn2, g2, bg2), (Wsn3, bsn3, bn3, g3, bg3))
    for Wsn, bsn, bn, g, bg in layer_refs:
        hsm = jnp.dot(h, Wsn[...], preferred_element_type=jnp.float32) + bsn[...]
        h_self = hsm[:, :H]
        m = hsm[:, H:]
        agg = jnp.dot(A, m, preferred_element_type=jnp.float32)
        o = h_self + agg * invd + bn[...] * dscale + h
        h = jax.nn.relu(o * (_INV * g[...]) + bg[...])

    rowmask = lax.broadcasted_iota(jnp.int32, (NP, H), 0) < N
    hg = jnp.sum(jnp.where(rowmask, h, 0.0), axis=0, keepdims=True) * (1.0 / N)
    hg = jax.nn.relu(jnp.dot(hg, Wd1[...], preferred_element_type=jnp.float32)
                     + bd1[...])
    hg = jax.nn.relu(jnp.dot(hg, Wd2[...], preferred_element_type=jnp.float32)
                     + bd2[...])
    out[pl.ds(pid, 1), :] = (jnp.sum(hg * w3r[...], axis=1, keepdims=True)
                             + bd3[...])


@jax.jit
def _run(xT, edges, flat_weights):
    full = lambda shape: pl.BlockSpec(shape, lambda i: (0,) * len(shape))
    in_specs = [full((NP, B)), full((2, E))]
    in_specs += [full(w.shape) for w in flat_weights]
    return pl.pallas_call(
        _gnn_kernel,
        grid=(B,),
        in_specs=in_specs,
        out_specs=pl.BlockSpec((B, 1), lambda i: (0, 0)),
        out_shape=jax.ShapeDtypeStruct((B, 1), jnp.float32),
        scratch_shapes=[pltpu.VMEM((NP, NP), jnp.float32)],
    )(xT, edges, *flat_weights)


def kernel(x, params, edge_index):
    xT = jnp.zeros((NP, B), jnp.float32).at[:N].set(x.T)

    enc = params["enc"]
    dec = params["dec"]
    flat = [enc[0][0].reshape(1, H), enc[0][1].reshape(1, H),
            enc[1][0], enc[1][1].reshape(1, H),
            enc[2][0], enc[2][1].reshape(1, H)]
    for lp in params["layers"]:
        Wsn = jnp.concatenate([lp["Ws"], lp["Wn"]], axis=1)
        bsn = jnp.concatenate([lp["bs"], jnp.zeros((H,), jnp.float32)]
                              ).reshape(1, 2 * H)
        flat += [Wsn, bsn, lp["bn"].reshape(1, H),
                 lp["g"].reshape(1, H), lp["b"].reshape(1, H)]
    flat += [dec[0][0], dec[0][1].reshape(1, H),
             dec[1][0], dec[1][1].reshape(1, H // 2),
             dec[2][0].reshape(1, H // 2), dec[2][1].reshape(1, 1)]

    return _run(xT, edge_index, tuple(flat))
